# baseline (speedup 1.0000x reference)
import functools

import jax
import jax.numpy as jnp
from jax import lax
from jax.experimental import pallas as pl
from jax.experimental.pallas import tpu as pltpu
from jax.experimental.pallas import tpu_sc as plsc

V = 1000001
D = 64
B = 4096
L = 200
N = B * L
NC = 2
NS = 16
NW = NC * NS
PER_W = N // NW
CH = 128
NCHUNK = PER_W // CH


def _sc_gather(table, ids):
    mesh = plsc.VectorSubcoreMesh(core_axis_name="c", subcore_axis_name="s")

    @functools.partial(
        pl.kernel,
        out_type=jax.ShapeDtypeStruct((N, D), jnp.float32),
        mesh=mesh,
        scratch_types=[
            pltpu.VMEM((PER_W,), jnp.int32),
            pltpu.VMEM((CH, D), jnp.float32),
            pltpu.SemaphoreType.DMA,
        ],
        compiler_params=pltpu.CompilerParams(use_tc_tiling_on_sc=False),
    )
    def k(t_hbm, idx_hbm, out_hbm, idx_v, buf, sem):
        wid = lax.axis_index("s") * NC + lax.axis_index("c")
        base = wid * PER_W
        pltpu.sync_copy(idx_hbm.at[pl.ds(base, PER_W)], idx_v)

        def body(j, carry):
            pltpu.async_copy(t_hbm.at[idx_v.at[pl.ds(j * CH, CH)]], buf, sem).wait()
            pltpu.sync_copy(buf, out_hbm.at[pl.ds(base + j * CH, CH)])
            return carry

        lax.fori_loop(0, NCHUNK, body, 0)

    return k(table, ids)


BLK2 = 2048
CE = BLK2 * 2 * D
G2 = (N * D) // CE


def _mm_body(x_ref, w1_ref, b1_ref, w2_ref, b2_ref, o_ref):
    z = jnp.zeros((D, D), jnp.float32)
    w1d = jnp.concatenate(
        [jnp.concatenate([w1_ref[...], z], axis=1),
         jnp.concatenate([z, w1_ref[...]], axis=1)], axis=0)
    w2d = jnp.concatenate(
        [jnp.concatenate([w2_ref[...], z], axis=1),
         jnp.concatenate([z, w2_ref[...]], axis=1)], axis=0)
    b1d = jnp.concatenate([b1_ref[...], b1_ref[...]], axis=1)
    b2d = jnp.concatenate([b2_ref[...], b2_ref[...]], axis=1)
    x = x_ref[...].reshape(BLK2, 2 * D)
    h = jnp.dot(x, w1d, preferred_element_type=jnp.float32) + b1d
    y = jnp.dot(h, w2d, preferred_element_type=jnp.float32) + b2d
    o_ref[...] = y.reshape(CE)


def _final_mm(xf, W1, b1, W2, b2):
    return pl.pallas_call(
        _mm_body,
        grid=(G2,),
        in_specs=[
            pl.BlockSpec((CE,), lambda i: (i,)),
            pl.BlockSpec((D, D), lambda i: (0, 0)),
            pl.BlockSpec((1, D), lambda i: (0, 0)),
            pl.BlockSpec((D, D), lambda i: (0, 0)),
            pl.BlockSpec((1, D), lambda i: (0, 0)),
        ],
        out_specs=pl.BlockSpec((CE,), lambda i: (i,)),
        out_shape=jax.ShapeDtypeStruct((N * D,), jnp.float32),
    )(xf, W1, b1.reshape(1, D), W2, b2.reshape(1, D))


def kernel(input_ids, emb_table, W1, b1, W2, b2):
    ids = input_ids.reshape(N)
    g = _sc_gather(emb_table, ids)
    yf = _final_mm(g.reshape(N * D), W1, b1, W2, b2)
    return (yf.reshape(B, L, D),)


# speedup vs baseline: 1.1363x; 1.1363x over previous
import functools

import jax
import jax.numpy as jnp
from jax import lax
from jax.experimental import pallas as pl
from jax.experimental.pallas import tpu as pltpu
from jax.experimental.pallas import tpu_sc as plsc

V = 1000001
D = 64
B = 4096
L = 200
N = B * L
NC = 2
NS = 16
NW = NC * NS
PER_W = N // NW
CH = 128
NCHUNK = PER_W // CH


def _sc_gather(table, ids):
    mesh = plsc.VectorSubcoreMesh(core_axis_name="c", subcore_axis_name="s")

    @functools.partial(
        pl.kernel,
        out_type=jax.ShapeDtypeStruct((N, D), jnp.float32),
        mesh=mesh,
        scratch_types=[
            pltpu.VMEM((PER_W,), jnp.int32),
            pltpu.VMEM((CH, D), jnp.float32),
            pltpu.VMEM((CH, D), jnp.float32),
            pltpu.SemaphoreType.DMA,
            pltpu.SemaphoreType.DMA,
            pltpu.SemaphoreType.DMA,
            pltpu.SemaphoreType.DMA,
        ],
        compiler_params=pltpu.CompilerParams(use_tc_tiling_on_sc=False),
    )
    def k(t_hbm, idx_hbm, out_hbm, idx_v, buf_a, buf_b, sg_a, sg_b, sw_a, sw_b):
        wid = lax.axis_index("s") * NC + lax.axis_index("c")
        base = wid * PER_W
        pltpu.sync_copy(idx_hbm.at[pl.ds(base, PER_W)], idx_v)

        def body(j, carry):
            ja = 2 * j
            jb = 2 * j + 1
            ga = pltpu.async_copy(
                t_hbm.at[idx_v.at[pl.ds(ja * CH, CH)]], buf_a, sg_a)
            gb = pltpu.async_copy(
                t_hbm.at[idx_v.at[pl.ds(jb * CH, CH)]], buf_b, sg_b)
            ga.wait()
            wa = pltpu.async_copy(
                buf_a, out_hbm.at[pl.ds(base + ja * CH, CH)], sw_a)
            gb.wait()
            wb = pltpu.async_copy(
                buf_b, out_hbm.at[pl.ds(base + jb * CH, CH)], sw_b)
            wa.wait()
            wb.wait()
            return carry

        lax.fori_loop(0, NCHUNK // 2, body, 0)

    return k(table, ids)


BLK2 = 8192
CE = BLK2 * 2 * D
G2 = (N * D) // CE


def _mm_body(x_ref, w1_ref, b1_ref, w2_ref, b2_ref, o_ref):
    z = jnp.zeros((D, D), jnp.float32)
    w1d = jnp.concatenate(
        [jnp.concatenate([w1_ref[...], z], axis=1),
         jnp.concatenate([z, w1_ref[...]], axis=1)], axis=0)
    w2d = jnp.concatenate(
        [jnp.concatenate([w2_ref[...], z], axis=1),
         jnp.concatenate([z, w2_ref[...]], axis=1)], axis=0)
    b1d = jnp.concatenate([b1_ref[...], b1_ref[...]], axis=1)
    b2d = jnp.concatenate([b2_ref[...], b2_ref[...]], axis=1)
    x = x_ref[...].reshape(BLK2, 2 * D)
    h = jnp.dot(x, w1d, preferred_element_type=jnp.float32) + b1d
    y = jnp.dot(h, w2d, preferred_element_type=jnp.float32) + b2d
    o_ref[...] = y.reshape(CE)


def _final_mm(xf, W1, b1, W2, b2):
    return pl.pallas_call(
        _mm_body,
        grid=(G2,),
        in_specs=[
            pl.BlockSpec((CE,), lambda i: (i,)),
            pl.BlockSpec((D, D), lambda i: (0, 0)),
            pl.BlockSpec((1, D), lambda i: (0, 0)),
            pl.BlockSpec((D, D), lambda i: (0, 0)),
            pl.BlockSpec((1, D), lambda i: (0, 0)),
        ],
        out_specs=pl.BlockSpec((CE,), lambda i: (i,)),
        out_shape=jax.ShapeDtypeStruct((N * D,), jnp.float32),
    )(xf, W1, b1.reshape(1, D), W2, b2.reshape(1, D))


def kernel(input_ids, emb_table, W1, b1, W2, b2):
    ids = input_ids.reshape(N)
    g = _sc_gather(emb_table, ids)
    yf = _final_mm(g.reshape(N * D), W1, b1, W2, b2)
    return (yf.reshape(B, L, D),)


# speedup vs baseline: 1.1672x; 1.0271x over previous
import functools

import jax
import jax.numpy as jnp
from jax import lax
from jax.experimental import pallas as pl
from jax.experimental.pallas import tpu as pltpu
from jax.experimental.pallas import tpu_sc as plsc

V = 1000001
D = 64
B = 4096
L = 200
N = B * L
NC = 2
NS = 16
NW = NC * NS
PER_W = N // NW
CH = 128
NCHUNK = PER_W // CH


def _sc_gather(table, ids):
    mesh = plsc.VectorSubcoreMesh(core_axis_name="c", subcore_axis_name="s")

    @functools.partial(
        pl.kernel,
        out_type=jax.ShapeDtypeStruct((N, D), jnp.float32),
        mesh=mesh,
        scratch_types=[
            pltpu.VMEM((PER_W,), jnp.int32),
            [pltpu.VMEM((CH, D), jnp.float32)] * 4,
            [pltpu.SemaphoreType.DMA] * 4,
            [pltpu.SemaphoreType.DMA] * 4,
        ],
        compiler_params=pltpu.CompilerParams(use_tc_tiling_on_sc=False),
    )
    def k(t_hbm, idx_hbm, out_hbm, idx_v, bufs, sgs, sws):
        wid = lax.axis_index("s") * NC + lax.axis_index("c")
        base = wid * PER_W
        pltpu.sync_copy(idx_hbm.at[pl.ds(base, PER_W)], idx_v)

        def body(j, carry):
            gathers = []
            for q in range(4):
                jq = 4 * j + q
                gathers.append(pltpu.async_copy(
                    t_hbm.at[idx_v.at[pl.ds(jq * CH, CH)]], bufs[q], sgs[q]))
            writes = []
            for q in range(4):
                jq = 4 * j + q
                gathers[q].wait()
                writes.append(pltpu.async_copy(
                    bufs[q], out_hbm.at[pl.ds(base + jq * CH, CH)], sws[q]))
            for q in range(4):
                writes[q].wait()
            return carry

        lax.fori_loop(0, NCHUNK // 4, body, 0)

    return k(table, ids)


BLK2 = 8192
CE = BLK2 * 2 * D
G2 = (N * D) // CE


def _mm_body(x_ref, w1_ref, b1_ref, w2_ref, b2_ref, o_ref):
    z = jnp.zeros((D, D), jnp.float32)
    w1d = jnp.concatenate(
        [jnp.concatenate([w1_ref[...], z], axis=1),
         jnp.concatenate([z, w1_ref[...]], axis=1)], axis=0)
    w2d = jnp.concatenate(
        [jnp.concatenate([w2_ref[...], z], axis=1),
         jnp.concatenate([z, w2_ref[...]], axis=1)], axis=0)
    b1d = jnp.concatenate([b1_ref[...], b1_ref[...]], axis=1)
    b2d = jnp.concatenate([b2_ref[...], b2_ref[...]], axis=1)
    x = x_ref[...].reshape(BLK2, 2 * D)
    h = jnp.dot(x, w1d, preferred_element_type=jnp.float32) + b1d
    y = jnp.dot(h, w2d, preferred_element_type=jnp.float32) + b2d
    o_ref[...] = y.reshape(CE)


def _final_mm(xf, W1, b1, W2, b2):
    return pl.pallas_call(
        _mm_body,
        grid=(G2,),
        in_specs=[
            pl.BlockSpec((CE,), lambda i: (i,)),
            pl.BlockSpec((D, D), lambda i: (0, 0)),
            pl.BlockSpec((1, D), lambda i: (0, 0)),
            pl.BlockSpec((D, D), lambda i: (0, 0)),
            pl.BlockSpec((1, D), lambda i: (0, 0)),
        ],
        out_specs=pl.BlockSpec((CE,), lambda i: (i,)),
        out_shape=jax.ShapeDtypeStruct((N * D,), jnp.float32),
    )(xf, W1, b1.reshape(1, D), W2, b2.reshape(1, D))


def kernel(input_ids, emb_table, W1, b1, W2, b2):
    ids = input_ids.reshape(N)
    g = _sc_gather(emb_table, ids)
    yf = _final_mm(g.reshape(N * D), W1, b1, W2, b2)
    return (yf.reshape(B, L, D),)
